# baseline (speedup 1.0000x reference)
import functools

import jax
import jax.numpy as jnp
from jax import lax
from jax.experimental import pallas as pl
from jax.experimental.pallas import tpu as pltpu
from jax.experimental.pallas import tpu_sc as plsc


def _argmin_body(nrows, ncodes, rows_per_tile, scale,
                 flat_ref, emb_ref, isqr_ref, esqr_ref,
                 idx_ref, loss_ref, embt_ref):
    i = pl.program_id(0)
    kb = embt_ref.shape[0]
    embt_ref[...] = emb_ref[:, pl.ds(i * kb, kb)].T
    z2 = flat_ref[...] * 2.0
    m2 = jax.lax.dot_general(z2, emb_ref[...], (((1,), (0,)), ((), ())),
                             preferred_element_type=jnp.float32)
    isqr = isqr_ref[...]
    ch = 128
    nchunks = ncodes // ch
    best_v = isqr - m2[:, 0:ch] + esqr_ref[:, 0:ch]
    best_c = jnp.zeros((rows_per_tile, ch), jnp.int32)
    for c in range(1, nchunks):
        dc = isqr - m2[:, c * ch:(c + 1) * ch] + esqr_ref[:, c * ch:(c + 1) * ch]
        lt = dc < best_v
        best_v = jnp.where(lt, dc, best_v)
        best_c = jnp.where(lt, c, best_c)
    lane = jax.lax.broadcasted_iota(jnp.int32, (rows_per_tile, ch), 1)
    best_j = best_c * ch + lane
    mn = jnp.min(best_v, axis=1, keepdims=True)
    idx_ref[...] = jnp.min(jnp.where(best_v == mn, best_j, ncodes), axis=1,
                           keepdims=True)
    part = jnp.sum(mn).reshape(1, 1)
    prev = jnp.where(i == 0, jnp.zeros((1, 1), jnp.float32), loss_ref[...])
    acc = prev + part
    last = i == (nrows // rows_per_tile) - 1
    loss_ref[...] = jnp.where(last, acc * scale, acc)


def _make_gather(nrows, emb_dim):
    info = plsc.get_sparse_core_info()
    nworkers = info.num_cores * info.num_subcores
    rows_per_w = nrows // nworkers
    mesh = plsc.VectorSubcoreMesh(core_axis_name="c", subcore_axis_name="s")

    @functools.partial(
        pl.kernel, mesh=mesh,
        out_type=jax.ShapeDtypeStruct((nrows, emb_dim), jnp.float32),
        scratch_types=[
            pltpu.VMEM((rows_per_w,), jnp.int32),
            pltpu.VMEM((rows_per_w, emb_dim), jnp.float32),
            pltpu.SemaphoreType.DMA,
        ],
        compiler_params=pltpu.CompilerParams(use_tc_tiling_on_sc=False),
    )
    def gather(table_hbm, idx_hbm, out_hbm, idx_v, rows_v, sem):
        wid = lax.axis_index("s") * info.num_cores + lax.axis_index("c")
        base = wid * rows_per_w
        pltpu.sync_copy(idx_hbm.at[pl.ds(base, rows_per_w)], idx_v)
        pltpu.async_copy(table_hbm.at[idx_v], rows_v, sem).wait()
        pltpu.sync_copy(rows_v, out_hbm.at[pl.ds(base, rows_per_w)])

    return gather


def kernel(z_latents, embeddings):
    commitment_cost = 0.25
    emb_dim = embeddings.shape[0]
    ncodes = embeddings.shape[1]
    z_shape = z_latents.shape
    flat = z_latents.reshape(-1, emb_dim)
    nrows = flat.shape[0]
    inputs_sqr = jnp.sum(flat ** 2, axis=1, keepdims=True)
    emb_sqr = jnp.sum(embeddings ** 2, axis=0, keepdims=True)

    rows = 1024
    grid = nrows // rows
    scale = (1.0 + commitment_cost) / (nrows * emb_dim)

    kb = ncodes // grid
    idx, loss, emb_t = pl.pallas_call(
        functools.partial(_argmin_body, nrows, ncodes, rows, scale),
        grid=(grid,),
        in_specs=[
            pl.BlockSpec((rows, emb_dim), lambda i: (i, 0)),
            pl.BlockSpec((emb_dim, ncodes), lambda i: (0, 0)),
            pl.BlockSpec((rows, 1), lambda i: (i, 0)),
            pl.BlockSpec((1, ncodes), lambda i: (0, 0)),
        ],
        out_specs=[
            pl.BlockSpec((rows, 1), lambda i: (i, 0)),
            pl.BlockSpec((1, 1), lambda i: (0, 0)),
            pl.BlockSpec((kb, emb_dim), lambda i: (i, 0)),
        ],
        out_shape=[
            jax.ShapeDtypeStruct((nrows, 1), jnp.int32),
            jax.ShapeDtypeStruct((1, 1), jnp.float32),
            jax.ShapeDtypeStruct((ncodes, emb_dim), jnp.float32),
        ],
    )(flat, embeddings, inputs_sqr, emb_sqr)

    q = _make_gather(nrows, emb_dim)(emb_t, idx.reshape(nrows))
    return q.reshape(z_shape), loss.reshape(())


# speedup vs baseline: 1.0142x; 1.0142x over previous
import functools

import jax
import jax.numpy as jnp
from jax import lax
from jax.experimental import pallas as pl
from jax.experimental.pallas import tpu as pltpu
from jax.experimental.pallas import tpu_sc as plsc


def _argmin_body(nrows, ncodes, rows_per_tile, scale,
                 flat_ref, emb_ref, isqr_ref, esqr_ref,
                 idx_ref, loss_ref):
    i = pl.program_id(0)
    z2 = flat_ref[...] * 2.0
    m2 = jax.lax.dot_general(z2, emb_ref[...], (((1,), (0,)), ((), ())),
                             preferred_element_type=jnp.float32)
    isqr = isqr_ref[...]
    ch = 128
    nchunks = ncodes // ch
    best_v = isqr - m2[:, 0:ch] + esqr_ref[:, 0:ch]
    best_c = jnp.zeros((rows_per_tile, ch), jnp.int32)
    for c in range(1, nchunks):
        dc = isqr - m2[:, c * ch:(c + 1) * ch] + esqr_ref[:, c * ch:(c + 1) * ch]
        lt = dc < best_v
        best_v = jnp.where(lt, dc, best_v)
        best_c = jnp.where(lt, c, best_c)
    lane = jax.lax.broadcasted_iota(jnp.int32, (rows_per_tile, ch), 1)
    best_j = best_c * ch + lane
    mn = jnp.min(best_v, axis=1, keepdims=True)
    idx_ref[...] = jnp.min(jnp.where(best_v == mn, best_j, ncodes), axis=1,
                           keepdims=True)
    part = jnp.sum(mn).reshape(1, 1)
    prev = jnp.where(i == 0, jnp.zeros((1, 1), jnp.float32), loss_ref[...])
    acc = prev + part
    last = i == (nrows // rows_per_tile) - 1
    loss_ref[...] = jnp.where(last, acc * scale, acc)


def _make_gather(nrows, emb_dim):
    info = plsc.get_sparse_core_info()
    nworkers = info.num_cores * info.num_subcores
    rows_per_w = nrows // nworkers
    mesh = plsc.VectorSubcoreMesh(core_axis_name="c", subcore_axis_name="s")

    @functools.partial(
        pl.kernel, mesh=mesh,
        out_type=jax.ShapeDtypeStruct((nrows, emb_dim), jnp.float32),
        scratch_types=[
            pltpu.VMEM((rows_per_w,), jnp.int32),
            pltpu.VMEM((rows_per_w, emb_dim), jnp.float32),
            pltpu.SemaphoreType.DMA,
        ],
        compiler_params=pltpu.CompilerParams(use_tc_tiling_on_sc=False),
    )
    def gather(table_hbm, idx_hbm, out_hbm, idx_v, rows_v, sem):
        wid = lax.axis_index("s") * info.num_cores + lax.axis_index("c")
        base = wid * rows_per_w
        pltpu.sync_copy(idx_hbm.at[pl.ds(base, rows_per_w)], idx_v)
        pltpu.async_copy(table_hbm.at[idx_v], rows_v, sem).wait()
        pltpu.sync_copy(rows_v, out_hbm.at[pl.ds(base, rows_per_w)])

    return gather


def kernel(z_latents, embeddings):
    commitment_cost = 0.25
    emb_dim = embeddings.shape[0]
    ncodes = embeddings.shape[1]
    z_shape = z_latents.shape
    flat = z_latents.reshape(-1, emb_dim)
    nrows = flat.shape[0]
    inputs_sqr = jnp.sum(flat ** 2, axis=1, keepdims=True)
    emb_sqr = jnp.sum(embeddings ** 2, axis=0, keepdims=True)
    emb_t = embeddings.T

    rows = 1024
    grid = nrows // rows
    scale = (1.0 + commitment_cost) / (nrows * emb_dim)

    idx, loss = pl.pallas_call(
        functools.partial(_argmin_body, nrows, ncodes, rows, scale),
        grid=(grid,),
        in_specs=[
            pl.BlockSpec((rows, emb_dim), lambda i: (i, 0)),
            pl.BlockSpec((emb_dim, ncodes), lambda i: (0, 0)),
            pl.BlockSpec((rows, 1), lambda i: (i, 0)),
            pl.BlockSpec((1, ncodes), lambda i: (0, 0)),
        ],
        out_specs=[
            pl.BlockSpec((rows, 1), lambda i: (i, 0)),
            pl.BlockSpec((1, 1), lambda i: (0, 0)),
        ],
        out_shape=[
            jax.ShapeDtypeStruct((nrows, 1), jnp.int32),
            jax.ShapeDtypeStruct((1, 1), jnp.float32),
        ],
    )(flat, embeddings, inputs_sqr, emb_sqr)

    q = _make_gather(nrows, emb_dim)(emb_t, idx.reshape(nrows))
    return q.reshape(z_shape), loss.reshape(())
